# initial kernel scaffold (unmeasured)
import jax
import jax.numpy as jnp
from jax import lax
from jax.experimental import pallas as pl
from jax.experimental.pallas import tpu as pltpu


def kernel(
    x,
):
    def body(*refs):
        pass

    out_shape = jax.ShapeDtypeStruct(..., jnp.float32)
    return pl.pallas_call(body, out_shape=out_shape)(...)


# baseline (device time: 454203 ns/iter reference)
import jax
import jax.numpy as jnp
from jax import lax
from jax.experimental import pallas as pl
from jax.experimental.pallas import tpu as pltpu

M, N = 16384, 1024
HALF = M // 2
CHUNK = 1024
N_CHUNK = HALF // CHUNK


def kernel(x):
    def body(x_hbm, out_ref, send_half, stage, obuf,
             stage_sems, load_sems, store_sems,
             y_send_sem, y_recv_sem, x_send_sem, x_recv_sem):
        mx = lax.axis_index("x")
        my = lax.axis_index("y")
        ypeer = (mx, 1 - my)
        xpeer = (1 - mx, my)
        h_base = mx * HALF
        o_base = (1 - mx) * HALF

        barrier = pltpu.get_barrier_semaphore()
        for nbr in (ypeer, xpeer):
            pl.semaphore_signal(barrier, inc=1, device_id=nbr,
                                device_id_type=pl.DeviceIdType.MESH)
        pl.semaphore_wait(barrier, 2)

        def start_x_load(base, c, slot):
            cp = pltpu.make_async_copy(
                x_hbm.at[pl.ds(base + c * CHUNK, CHUNK), :],
                stage.at[slot],
                stage_sems.at[slot],
            )
            cp.start()
            return cp

        cps = [None] * N_CHUNK
        cps[0] = start_x_load(h_base, 0, 0)
        if N_CHUNK > 1:
            cps[1] = start_x_load(h_base, 1, 1)
        for c in range(N_CHUNK):
            cps[c].wait()
            send_half[pl.ds(c * CHUNK, CHUNK), :] = (
                stage[c % 2].astype(jnp.bfloat16)
            )
            if c + 2 < N_CHUNK:
                cps[c + 2] = start_x_load(h_base, c + 2, c % 2)

        y_rdma = pltpu.make_async_remote_copy(
            src_ref=send_half,
            dst_ref=out_ref.at[pl.ds(h_base, HALF), :],
            send_sem=y_send_sem,
            recv_sem=y_recv_sem,
            device_id=ypeer,
            device_id_type=pl.DeviceIdType.MESH,
        )
        y_rdma.start()
        y_rdma.wait()

        x_rdma = pltpu.make_async_remote_copy(
            src_ref=out_ref.at[pl.ds(h_base, HALF), :],
            dst_ref=out_ref.at[pl.ds(h_base, HALF), :],
            send_sem=x_send_sem,
            recv_sem=x_recv_sem,
            device_id=xpeer,
            device_id_type=pl.DeviceIdType.MESH,
        )
        x_rdma.start()
        x_rdma.wait()

        def add_chunk(row, slot, local_bf16):
            ld = pltpu.make_async_copy(
                out_ref.at[pl.ds(row, CHUNK), :],
                obuf.at[slot],
                load_sems.at[slot],
            )
            ld.start()
            ld.wait()
            obuf[slot] = obuf[slot] + local_bf16
            st = pltpu.make_async_copy(
                obuf.at[slot],
                out_ref.at[pl.ds(row, CHUNK), :],
                store_sems.at[slot],
            )
            st.start()
            st.wait()

        for c in range(N_CHUNK):
            add_chunk(h_base + c * CHUNK, c % 2,
                      send_half[pl.ds(c * CHUNK, CHUNK), :])
        cps[0] = start_x_load(o_base, 0, 0)
        if N_CHUNK > 1:
            cps[1] = start_x_load(o_base, 1, 1)
        for c in range(N_CHUNK):
            cps[c].wait()
            local_bf16 = stage[c % 2].astype(jnp.bfloat16)
            if c + 2 < N_CHUNK:
                cps[c + 2] = start_x_load(o_base, c + 2, c % 2)
            add_chunk(o_base + c * CHUNK, c % 2, local_bf16)

    return pl.pallas_call(
        body,
        out_shape=jax.ShapeDtypeStruct((M, N), jnp.bfloat16),
        in_specs=[pl.BlockSpec(memory_space=pl.ANY)],
        out_specs=pl.BlockSpec(memory_space=pl.ANY),
        scratch_shapes=[
            pltpu.VMEM((HALF, N), jnp.bfloat16),
            pltpu.VMEM((2, CHUNK, N), jnp.float32),
            pltpu.VMEM((2, CHUNK, N), jnp.bfloat16),
            pltpu.SemaphoreType.DMA((2,)),
            pltpu.SemaphoreType.DMA((2,)),
            pltpu.SemaphoreType.DMA((2,)),
            pltpu.SemaphoreType.DMA,
            pltpu.SemaphoreType.DMA,
            pltpu.SemaphoreType.DMA,
            pltpu.SemaphoreType.DMA,
        ],
        compiler_params=pltpu.CompilerParams(collective_id=0),
    )(x)


# device time: 288247 ns/iter; 1.5757x vs baseline; 1.5757x over previous
import jax
import jax.numpy as jnp
from jax import lax
from jax.experimental import pallas as pl
from jax.experimental.pallas import tpu as pltpu

M, N = 16384, 1024
HALF = M // 2
CHUNK = 1024
NC = HALF // CHUNK


def kernel(x):
    def body(x_hbm, out_ref, send_half, stage, obuf,
             stage_sems, load_sems, store_sems,
             y_send_sems, y_recv_sems, x_send_sems, x_recv_sems):
        mx = lax.axis_index("x")
        my = lax.axis_index("y")
        ypeer = (mx, 1 - my)
        xpeer = (1 - mx, my)
        h_base = mx * HALF
        o_base = (1 - mx) * HALF

        barrier = pltpu.get_barrier_semaphore()
        for nbr in (ypeer, xpeer):
            pl.semaphore_signal(barrier, inc=1, device_id=nbr,
                                device_id_type=pl.DeviceIdType.MESH)
        pl.semaphore_wait(barrier, 2)

        def start_x_load(base, c, slot):
            cp = pltpu.make_async_copy(
                x_hbm.at[pl.ds(base + c * CHUNK, CHUNK), :],
                stage.at[slot],
                stage_sems.at[slot],
            )
            cp.start()
            return cp

        y_rdmas = [None] * NC
        cps = [None] * NC
        cps[0] = start_x_load(h_base, 0, 0)
        if NC > 1:
            cps[1] = start_x_load(h_base, 1, 1)
        for c in range(NC):
            cps[c].wait()
            send_half[pl.ds(c * CHUNK, CHUNK), :] = (
                stage[c % 2].astype(jnp.bfloat16)
            )
            if c + 2 < NC:
                cps[c + 2] = start_x_load(h_base, c + 2, c % 2)
            y_rdmas[c] = pltpu.make_async_remote_copy(
                src_ref=send_half.at[pl.ds(c * CHUNK, CHUNK), :],
                dst_ref=out_ref.at[pl.ds(h_base + c * CHUNK, CHUNK), :],
                send_sem=y_send_sems.at[c],
                recv_sem=y_recv_sems.at[c],
                device_id=ypeer,
                device_id_type=pl.DeviceIdType.MESH,
            )
            y_rdmas[c].start()
        x_rdmas = [None] * NC
        for c in range(NC):
            y_rdmas[c].wait_recv()
            x_rdmas[c] = pltpu.make_async_remote_copy(
                src_ref=out_ref.at[pl.ds(h_base + c * CHUNK, CHUNK), :],
                dst_ref=out_ref.at[pl.ds(h_base + c * CHUNK, CHUNK), :],
                send_sem=x_send_sems.at[c],
                recv_sem=x_recv_sems.at[c],
                device_id=xpeer,
                device_id_type=pl.DeviceIdType.MESH,
            )
            x_rdmas[c].start()
        for c in range(NC):
            y_rdmas[c].wait_send()
            x_rdmas[c].wait()

        def add_chunk(row, slot, local_bf16):
            ld = pltpu.make_async_copy(
                out_ref.at[pl.ds(row, CHUNK), :],
                obuf.at[slot],
                load_sems.at[slot],
            )
            ld.start()
            ld.wait()
            obuf[slot] = obuf[slot] + local_bf16
            st = pltpu.make_async_copy(
                obuf.at[slot],
                out_ref.at[pl.ds(row, CHUNK), :],
                store_sems.at[slot],
            )
            st.start()
            st.wait()

        for c in range(NC):
            add_chunk(h_base + c * CHUNK, c % 2,
                      send_half[pl.ds(c * CHUNK, CHUNK), :])
        cps[0] = start_x_load(o_base, 0, 0)
        if NC > 1:
            cps[1] = start_x_load(o_base, 1, 1)
        for c in range(NC):
            cps[c].wait()
            local_bf16 = stage[c % 2].astype(jnp.bfloat16)
            if c + 2 < NC:
                cps[c + 2] = start_x_load(o_base, c + 2, c % 2)
            add_chunk(o_base + c * CHUNK, c % 2, local_bf16)

    return pl.pallas_call(
        body,
        out_shape=jax.ShapeDtypeStruct((M, N), jnp.bfloat16),
        in_specs=[pl.BlockSpec(memory_space=pl.ANY)],
        out_specs=pl.BlockSpec(memory_space=pl.ANY),
        scratch_shapes=[
            pltpu.VMEM((HALF, N), jnp.bfloat16),
            pltpu.VMEM((2, CHUNK, N), jnp.float32),
            pltpu.VMEM((2, CHUNK, N), jnp.bfloat16),
            pltpu.SemaphoreType.DMA((2,)),
            pltpu.SemaphoreType.DMA((2,)),
            pltpu.SemaphoreType.DMA((2,)),
            pltpu.SemaphoreType.DMA((NC,)),
            pltpu.SemaphoreType.DMA((NC,)),
            pltpu.SemaphoreType.DMA((NC,)),
            pltpu.SemaphoreType.DMA((NC,)),
        ],
        compiler_params=pltpu.CompilerParams(collective_id=0),
    )(x)


# device time: 242523 ns/iter; 1.8728x vs baseline; 1.1885x over previous
import jax
import jax.numpy as jnp
from jax import lax
from jax.experimental import pallas as pl
from jax.experimental.pallas import tpu as pltpu

M, N = 16384, 1024
HALF = M // 2
CHUNK = 1024
NC = HALF // CHUNK


def kernel(x):
    def body(x_hbm, out_ref, send_half, stage, obuf,
             stage_sems, load_sems, store_sems,
             y_send_sems, y_recv_sems, x_send_sems, x_recv_sems):
        mx = lax.axis_index("x")
        my = lax.axis_index("y")
        ypeer = (mx, 1 - my)
        xpeer = (1 - mx, my)
        h_base = mx * HALF
        o_base = (1 - mx) * HALF

        barrier = pltpu.get_barrier_semaphore()
        for nbr in (ypeer, xpeer):
            pl.semaphore_signal(barrier, inc=1, device_id=nbr,
                                device_id_type=pl.DeviceIdType.MESH)
        pl.semaphore_wait(barrier, 2)

        def start_x_load(base, c, slot):
            cp = pltpu.make_async_copy(
                x_hbm.at[pl.ds(base + c * CHUNK, CHUNK), :],
                stage.at[slot],
                stage_sems.at[slot],
            )
            cp.start()
            return cp

        y_rdmas = [None] * NC
        x_rdmas = [None] * NC
        cps = [None] * NC
        cps[0] = start_x_load(h_base, 0, 0)
        if NC > 1:
            cps[1] = start_x_load(h_base, 1, 1)
        for c in range(NC):
            cps[c].wait()
            send_half[pl.ds(c * CHUNK, CHUNK), :] = (
                stage[c % 2].astype(jnp.bfloat16)
            )
            if c + 2 < NC:
                cps[c + 2] = start_x_load(h_base, c + 2, c % 2)
            y_rdmas[c] = pltpu.make_async_remote_copy(
                src_ref=send_half.at[pl.ds(c * CHUNK, CHUNK), :],
                dst_ref=out_ref.at[pl.ds(h_base + c * CHUNK, CHUNK), :],
                send_sem=y_send_sems.at[c],
                recv_sem=y_recv_sems.at[c],
                device_id=ypeer,
                device_id_type=pl.DeviceIdType.MESH,
            )
            y_rdmas[c].start()

        def add_chunk(row, slot, local_bf16):
            ld = pltpu.make_async_copy(
                out_ref.at[pl.ds(row, CHUNK), :],
                obuf.at[slot],
                load_sems.at[slot],
            )
            ld.start()
            ld.wait()
            obuf[slot] = obuf[slot] + local_bf16
            st = pltpu.make_async_copy(
                obuf.at[slot],
                out_ref.at[pl.ds(row, CHUNK), :],
                store_sems.at[slot],
            )
            st.start()
            st.wait()

        def direct_add(d):
            y_rdmas[d].wait_send()
            x_rdmas[d].wait_send()
            add_chunk(h_base + d * CHUNK, d % 2,
                      send_half[pl.ds(d * CHUNK, CHUNK), :])

        def other_add(o):
            cps[o].wait()
            x_rdmas[o].wait_recv()
            add_chunk(o_base + o * CHUNK, o % 2,
                      stage[o % 2].astype(jnp.bfloat16))

        for c in range(NC):
            if c >= 2:
                other_add(c - 2)
            cps[c] = start_x_load(o_base, c, c % 2)
            y_rdmas[c].wait_recv()
            x_rdmas[c] = pltpu.make_async_remote_copy(
                src_ref=out_ref.at[pl.ds(h_base + c * CHUNK, CHUNK), :],
                dst_ref=out_ref.at[pl.ds(h_base + c * CHUNK, CHUNK), :],
                send_sem=x_send_sems.at[c],
                recv_sem=x_recv_sems.at[c],
                device_id=xpeer,
                device_id_type=pl.DeviceIdType.MESH,
            )
            x_rdmas[c].start()
            if c >= 1:
                direct_add(c - 1)

        direct_add(NC - 1)
        for o in range(max(0, NC - 2), NC):
            other_add(o)

    return pl.pallas_call(
        body,
        out_shape=jax.ShapeDtypeStruct((M, N), jnp.bfloat16),
        in_specs=[pl.BlockSpec(memory_space=pl.ANY)],
        out_specs=pl.BlockSpec(memory_space=pl.ANY),
        scratch_shapes=[
            pltpu.VMEM((HALF, N), jnp.bfloat16),
            pltpu.VMEM((2, CHUNK, N), jnp.float32),
            pltpu.VMEM((2, CHUNK, N), jnp.bfloat16),
            pltpu.SemaphoreType.DMA((2,)),
            pltpu.SemaphoreType.DMA((2,)),
            pltpu.SemaphoreType.DMA((2,)),
            pltpu.SemaphoreType.DMA((NC,)),
            pltpu.SemaphoreType.DMA((NC,)),
            pltpu.SemaphoreType.DMA((NC,)),
            pltpu.SemaphoreType.DMA((NC,)),
        ],
        compiler_params=pltpu.CompilerParams(collective_id=0),
    )(x)


# device time: 228161 ns/iter; 1.9907x vs baseline; 1.0629x over previous
import jax
import jax.numpy as jnp
from jax import lax
from jax.experimental import pallas as pl
from jax.experimental.pallas import tpu as pltpu

M, N = 16384, 1024
HALF = M // 2
CHUNK = 512
NC = HALF // CHUNK


def kernel(x):
    def body(x_hbm, out_ref, send_half, stage, obuf,
             stage_sems, load_sems, store_sems,
             y_send_sems, y_recv_sems, x_send_sems, x_recv_sems):
        mx = lax.axis_index("x")
        my = lax.axis_index("y")
        ypeer = (mx, 1 - my)
        xpeer = (1 - mx, my)
        h_base = mx * HALF
        o_base = (1 - mx) * HALF

        barrier = pltpu.get_barrier_semaphore()
        for nbr in (ypeer, xpeer):
            pl.semaphore_signal(barrier, inc=1, device_id=nbr,
                                device_id_type=pl.DeviceIdType.MESH)
        pl.semaphore_wait(barrier, 2)

        def start_x_load(base, c, slot):
            cp = pltpu.make_async_copy(
                x_hbm.at[pl.ds(base + c * CHUNK, CHUNK), :],
                stage.at[slot],
                stage_sems.at[slot],
            )
            cp.start()
            return cp

        y_rdmas = [None] * NC
        x_rdmas = [None] * NC
        cps = [None] * NC
        cps[0] = start_x_load(h_base, 0, 0)
        if NC > 1:
            cps[1] = start_x_load(h_base, 1, 1)
        for c in range(NC):
            cps[c].wait()
            send_half[pl.ds(c * CHUNK, CHUNK), :] = (
                stage[c % 2].astype(jnp.bfloat16)
            )
            if c + 2 < NC:
                cps[c + 2] = start_x_load(h_base, c + 2, c % 2)
            y_rdmas[c] = pltpu.make_async_remote_copy(
                src_ref=send_half.at[pl.ds(c * CHUNK, CHUNK), :],
                dst_ref=out_ref.at[pl.ds(h_base + c * CHUNK, CHUNK), :],
                send_sem=y_send_sems.at[c],
                recv_sem=y_recv_sems.at[c],
                device_id=ypeer,
                device_id_type=pl.DeviceIdType.MESH,
            )
            y_rdmas[c].start()

        def add_chunk(row, slot, local_bf16):
            ld = pltpu.make_async_copy(
                out_ref.at[pl.ds(row, CHUNK), :],
                obuf.at[slot],
                load_sems.at[slot],
            )
            ld.start()
            ld.wait()
            obuf[slot] = obuf[slot] + local_bf16
            st = pltpu.make_async_copy(
                obuf.at[slot],
                out_ref.at[pl.ds(row, CHUNK), :],
                store_sems.at[slot],
            )
            st.start()
            st.wait()

        def direct_add(d):
            y_rdmas[d].wait_send()
            x_rdmas[d].wait_send()
            add_chunk(h_base + d * CHUNK, d % 2,
                      send_half[pl.ds(d * CHUNK, CHUNK), :])

        def other_add(o):
            cps[o].wait()
            x_rdmas[o].wait_recv()
            add_chunk(o_base + o * CHUNK, o % 2,
                      stage[o % 2].astype(jnp.bfloat16))

        for c in range(NC):
            if c >= 2:
                other_add(c - 2)
            cps[c] = start_x_load(o_base, c, c % 2)
            y_rdmas[c].wait_recv()
            x_rdmas[c] = pltpu.make_async_remote_copy(
                src_ref=out_ref.at[pl.ds(h_base + c * CHUNK, CHUNK), :],
                dst_ref=out_ref.at[pl.ds(h_base + c * CHUNK, CHUNK), :],
                send_sem=x_send_sems.at[c],
                recv_sem=x_recv_sems.at[c],
                device_id=xpeer,
                device_id_type=pl.DeviceIdType.MESH,
            )
            x_rdmas[c].start()
            if c >= 1:
                direct_add(c - 1)

        direct_add(NC - 1)
        for o in range(max(0, NC - 2), NC):
            other_add(o)

    return pl.pallas_call(
        body,
        out_shape=jax.ShapeDtypeStruct((M, N), jnp.bfloat16),
        in_specs=[pl.BlockSpec(memory_space=pl.ANY)],
        out_specs=pl.BlockSpec(memory_space=pl.ANY),
        scratch_shapes=[
            pltpu.VMEM((HALF, N), jnp.bfloat16),
            pltpu.VMEM((2, CHUNK, N), jnp.float32),
            pltpu.VMEM((2, CHUNK, N), jnp.bfloat16),
            pltpu.SemaphoreType.DMA((2,)),
            pltpu.SemaphoreType.DMA((2,)),
            pltpu.SemaphoreType.DMA((2,)),
            pltpu.SemaphoreType.DMA((NC,)),
            pltpu.SemaphoreType.DMA((NC,)),
            pltpu.SemaphoreType.DMA((NC,)),
            pltpu.SemaphoreType.DMA((NC,)),
        ],
        compiler_params=pltpu.CompilerParams(collective_id=0),
    )(x)


# device time: 225749 ns/iter; 2.0120x vs baseline; 1.0107x over previous
import jax
import jax.numpy as jnp
from jax import lax
from jax.experimental import pallas as pl
from jax.experimental.pallas import tpu as pltpu

M, N = 16384, 1024
HALF = M // 2
CHUNK = 512
NC = HALF // CHUNK


def kernel(x):
    def body(x_hbm, out_ref, send_half, stage, obuf,
             stage_sems, load_sems, store_sems,
             y_send_sems, y_recv_sems, x_send_sems, x_recv_sems):
        mx = lax.axis_index("x")
        my = lax.axis_index("y")
        ypeer = (mx, 1 - my)
        xpeer = (1 - mx, my)
        h_base = mx * HALF
        o_base = (1 - mx) * HALF

        def drow(c):
            return pl.ds(h_base + c * CHUNK, CHUNK)

        def orow(c):
            return pl.ds(o_base + c * CHUNK, CHUNK)

        barrier = pltpu.get_barrier_semaphore()
        for nbr in (ypeer, xpeer):
            pl.semaphore_signal(barrier, inc=1, device_id=nbr,
                                device_id_type=pl.DeviceIdType.MESH)
        pl.semaphore_wait(barrier, 2)

        def start_x_load(base, c, slot):
            cp = pltpu.make_async_copy(
                x_hbm.at[pl.ds(base + c * CHUNK, CHUNK), :],
                stage.at[slot],
                stage_sems.at[slot],
            )
            cp.start()
            return cp

        y_rdmas = [None] * NC
        x_rdmas = [None] * NC
        cps = [None] * NC
        cps[0] = start_x_load(h_base, 0, 0)
        if NC > 1:
            cps[1] = start_x_load(h_base, 1, 1)
        for c in range(NC):
            cps[c].wait()
            send_half[pl.ds(c * CHUNK, CHUNK), :] = (
                stage[c % 2].astype(jnp.bfloat16)
            )
            if c + 2 < NC:
                cps[c + 2] = start_x_load(h_base, c + 2, c % 2)
            y_rdmas[c] = pltpu.make_async_remote_copy(
                src_ref=send_half.at[pl.ds(c * CHUNK, CHUNK), :],
                dst_ref=out_ref.at[drow(c)],
                send_sem=y_send_sems.at[c],
                recv_sem=y_recv_sems.at[c],
                device_id=ypeer,
                device_id_type=pl.DeviceIdType.MESH,
            )
            y_rdmas[c].start()

        ld_d = [None] * NC
        st_d = [None] * NC
        ld_o = [None] * NC
        st_o = [None] * NC
        cps2 = [None] * NC

        def finish_direct(d):
            ld_d[d].wait()
            obuf[d % 2] = obuf[d % 2] + send_half[pl.ds(d * CHUNK, CHUNK), :]
            y_rdmas[d].wait_send()
            x_rdmas[d].wait_send()
            st_d[d] = pltpu.make_async_copy(
                obuf.at[d % 2], out_ref.at[drow(d)], store_sems.at[d % 2])
            st_d[d].start()

        def finish_other(o):
            oslot = 2 + (o % 2)
            x_rdmas[o].wait_recv()
            if o >= 2:
                st_o[o - 2].wait()
            ld_o[o] = pltpu.make_async_copy(
                out_ref.at[orow(o)], obuf.at[oslot], load_sems.at[oslot])
            ld_o[o].start()
            ld_o[o].wait()
            cps2[o].wait()
            obuf[oslot] = obuf[oslot] + stage[o % 2].astype(jnp.bfloat16)
            st_o[o] = pltpu.make_async_copy(
                obuf.at[oslot], out_ref.at[orow(o)], store_sems.at[oslot])
            st_o[o].start()

        for c in range(NC):
            y_rdmas[c].wait_recv()
            x_rdmas[c] = pltpu.make_async_remote_copy(
                src_ref=out_ref.at[drow(c)],
                dst_ref=out_ref.at[drow(c)],
                send_sem=x_send_sems.at[c],
                recv_sem=x_recv_sems.at[c],
                device_id=xpeer,
                device_id_type=pl.DeviceIdType.MESH,
            )
            x_rdmas[c].start()
            if c >= 2:
                st_d[c - 2].wait()
            ld_d[c] = pltpu.make_async_copy(
                out_ref.at[drow(c)], obuf.at[c % 2], load_sems.at[c % 2])
            ld_d[c].start()
            if c >= 1:
                finish_direct(c - 1)
            if c >= 2:
                finish_other(c - 2)
            cps2[c] = start_x_load(o_base, c, c % 2)

        finish_direct(NC - 1)
        for o in (NC - 2, NC - 1):
            finish_other(o)
        st_d[NC - 2].wait()
        st_d[NC - 1].wait()
        st_o[NC - 2].wait()
        st_o[NC - 1].wait()

    return pl.pallas_call(
        body,
        out_shape=jax.ShapeDtypeStruct((M, N), jnp.bfloat16),
        in_specs=[pl.BlockSpec(memory_space=pl.ANY)],
        out_specs=pl.BlockSpec(memory_space=pl.ANY),
        scratch_shapes=[
            pltpu.VMEM((HALF, N), jnp.bfloat16),
            pltpu.VMEM((2, CHUNK, N), jnp.float32),
            pltpu.VMEM((4, CHUNK, N), jnp.bfloat16),
            pltpu.SemaphoreType.DMA((2,)),
            pltpu.SemaphoreType.DMA((4,)),
            pltpu.SemaphoreType.DMA((4,)),
            pltpu.SemaphoreType.DMA((NC,)),
            pltpu.SemaphoreType.DMA((NC,)),
            pltpu.SemaphoreType.DMA((NC,)),
            pltpu.SemaphoreType.DMA((NC,)),
        ],
        compiler_params=pltpu.CompilerParams(collective_id=0),
    )(x)
